# Initial kernel scaffold; baseline (speedup 1.0000x reference)
import jax
import jax.numpy as jnp
from jax.experimental import pallas as pl


def kernel(z, m, W1, b1, W2, b2):
    raise NotImplementedError("write your pallas kernel here")


# speedup vs baseline: 1.6919x; 1.6919x over previous
import jax
import jax.numpy as jnp
from jax.experimental import pallas as pl

DIM = 1024
N_OPT = 17
TOK_BLK = 512


def _fused_mlp(z_ref, m_ref, w1z_ref, w1m_ref, b1_ref, w2_ref, b2_ref, o_ref):
    zb = z_ref[...].astype(jnp.bfloat16)
    mb = m_ref[...].astype(jnp.bfloat16)
    h = jnp.dot(zb, w1z_ref[...], preferred_element_type=jnp.float32)
    h = h + jnp.dot(mb, w1m_ref[...], preferred_element_type=jnp.float32)
    h = h + b1_ref[...]
    h = 0.5 * h * (1.0 + jax.lax.erf(h * 0.7071067811865476))
    out = jnp.dot(h, w2_ref[...], preferred_element_type=jnp.float32)
    o_ref[...] = out + b2_ref[...]


def kernel(z, m, W1, b1, W2, b2):
    n_tok = z.shape[0]
    w1z = W1[:, :DIM].T.astype(jnp.bfloat16)
    w1m = W1[:, DIM:].T.astype(jnp.bfloat16)
    w2t = W2.T
    b1r = b1.reshape(1, DIM)
    b2r = b2.reshape(1, N_OPT)

    grid = (n_tok // TOK_BLK,)
    return pl.pallas_call(
        _fused_mlp,
        grid=grid,
        in_specs=[
            pl.BlockSpec((TOK_BLK, DIM), lambda i: (i, 0)),
            pl.BlockSpec((TOK_BLK, DIM), lambda i: (i, 0)),
            pl.BlockSpec((DIM, DIM), lambda i: (0, 0)),
            pl.BlockSpec((DIM, DIM), lambda i: (0, 0)),
            pl.BlockSpec((1, DIM), lambda i: (0, 0)),
            pl.BlockSpec((DIM, N_OPT), lambda i: (0, 0)),
            pl.BlockSpec((1, N_OPT), lambda i: (0, 0)),
        ],
        out_specs=pl.BlockSpec((TOK_BLK, N_OPT), lambda i: (i, 0)),
        out_shape=jax.ShapeDtypeStruct((n_tok, N_OPT), jnp.float32),
    )(z, m, w1z, w1m, b1r, w2t, b2r)


# speedup vs baseline: 1.7939x; 1.0603x over previous
import jax
import jax.numpy as jnp
from jax.experimental import pallas as pl
from jax.experimental.pallas import tpu as pltpu

DIM = 1024
N_OPT = 17
TOK_BLK = 512

_DN = (((1,), (1,)), ((), ()))


def _fused_mlp(z_ref, m_ref, w1_ref, b1_ref, w2_ref, b2_ref, o_ref, w1_bf):
    @pl.when(pl.program_id(0) == 0)
    def _cast_weights():
        w1_bf[...] = w1_ref[...].astype(jnp.bfloat16)

    zb = z_ref[...].astype(jnp.bfloat16)
    mb = m_ref[...].astype(jnp.bfloat16)
    h = jax.lax.dot_general(zb, w1_bf[:, :DIM], _DN,
                            preferred_element_type=jnp.float32)
    h = h + jax.lax.dot_general(mb, w1_bf[:, DIM:], _DN,
                                preferred_element_type=jnp.float32)
    h = h + b1_ref[...]
    h = 0.5 * h * (1.0 + jax.lax.erf(h * 0.7071067811865476))
    out = jnp.dot(h, w2_ref[...], preferred_element_type=jnp.float32)
    o_ref[...] = out + b2_ref[...]


def kernel(z, m, W1, b1, W2, b2):
    n_tok = z.shape[0]
    w2t = W2.T
    b1r = b1.reshape(1, DIM)
    b2r = b2.reshape(1, N_OPT)

    grid = (n_tok // TOK_BLK,)
    return pl.pallas_call(
        _fused_mlp,
        grid=grid,
        in_specs=[
            pl.BlockSpec((TOK_BLK, DIM), lambda i: (i, 0)),
            pl.BlockSpec((TOK_BLK, DIM), lambda i: (i, 0)),
            pl.BlockSpec((DIM, 2 * DIM), lambda i: (0, 0)),
            pl.BlockSpec((1, DIM), lambda i: (0, 0)),
            pl.BlockSpec((DIM, N_OPT), lambda i: (0, 0)),
            pl.BlockSpec((1, N_OPT), lambda i: (0, 0)),
        ],
        out_specs=pl.BlockSpec((TOK_BLK, N_OPT), lambda i: (i, 0)),
        out_shape=jax.ShapeDtypeStruct((n_tok, N_OPT), jnp.float32),
        scratch_shapes=[pltpu.VMEM((DIM, 2 * DIM), jnp.bfloat16)],
    )(z, m, W1, b1r, w2t, b2r)


# speedup vs baseline: 1.8929x; 1.0552x over previous
import jax
import jax.numpy as jnp
from jax.experimental import pallas as pl
from jax.experimental.pallas import tpu as pltpu

DIM = 1024
N_OPT = 17
TOK_BLK = 1024

_DN = (((1,), (1,)), ((), ()))


def _fused_mlp(z_ref, m_ref, w1_ref, b1_ref, w2_ref, b2_ref, o_ref, w1_bf):
    @pl.when(pl.program_id(0) == 0)
    def _cast_weights():
        w1_bf[...] = w1_ref[...].astype(jnp.bfloat16)

    zb = z_ref[...].astype(jnp.bfloat16)
    mb = m_ref[...].astype(jnp.bfloat16)
    h = jax.lax.dot_general(zb, w1_bf[:, :DIM], _DN,
                            preferred_element_type=jnp.float32)
    h = h + jax.lax.dot_general(mb, w1_bf[:, DIM:], _DN,
                                preferred_element_type=jnp.float32)
    h = h + b1_ref[...]
    h = 0.5 * h * (1.0 + jax.lax.erf(h * 0.7071067811865476))
    out = jnp.dot(h, w2_ref[...], preferred_element_type=jnp.float32)
    o_ref[...] = out + b2_ref[...]


def kernel(z, m, W1, b1, W2, b2):
    n_tok = z.shape[0]
    w2t = W2.T
    b1r = b1.reshape(1, DIM)
    b2r = b2.reshape(1, N_OPT)

    grid = (n_tok // TOK_BLK,)
    return pl.pallas_call(
        _fused_mlp,
        grid=grid,
        in_specs=[
            pl.BlockSpec((TOK_BLK, DIM), lambda i: (i, 0)),
            pl.BlockSpec((TOK_BLK, DIM), lambda i: (i, 0)),
            pl.BlockSpec((DIM, 2 * DIM), lambda i: (0, 0)),
            pl.BlockSpec((1, DIM), lambda i: (0, 0)),
            pl.BlockSpec((DIM, N_OPT), lambda i: (0, 0)),
            pl.BlockSpec((1, N_OPT), lambda i: (0, 0)),
        ],
        out_specs=pl.BlockSpec((TOK_BLK, N_OPT), lambda i: (i, 0)),
        out_shape=jax.ShapeDtypeStruct((n_tok, N_OPT), jnp.float32),
        scratch_shapes=[pltpu.VMEM((DIM, 2 * DIM), jnp.bfloat16)],
    )(z, m, W1, b1r, w2t, b2r)


# speedup vs baseline: 1.9128x; 1.0105x over previous
import jax
import jax.numpy as jnp
from jax.experimental import pallas as pl
from jax.experimental.pallas import tpu as pltpu

DIM = 1024
N_OPT = 17
TOK_BLK = 1024

_DN = (((1,), (1,)), ((), ()))


def _fused_mlp(z_ref, m_ref, w1_ref, b1_ref, w2_ref, b2_ref, o_ref, w1_bf):
    w1_bf[...] = w1_ref[...].astype(jnp.bfloat16)

    zb = z_ref[...].astype(jnp.bfloat16)
    mb = m_ref[...].astype(jnp.bfloat16)
    h = jax.lax.dot_general(zb, w1_bf[:, :DIM], _DN,
                            preferred_element_type=jnp.float32)
    h = h + jax.lax.dot_general(mb, w1_bf[:, DIM:], _DN,
                                preferred_element_type=jnp.float32)
    h = h + b1_ref[...]
    h = 0.5 * h * (1.0 + jax.lax.erf(h * 0.7071067811865476))
    out = jnp.dot(h, w2_ref[...], preferred_element_type=jnp.float32)
    o_ref[...] = out + b2_ref[...]


def kernel(z, m, W1, b1, W2, b2):
    n_tok = z.shape[0]
    w2t = W2.T
    b1r = b1.reshape(1, DIM)
    b2r = b2.reshape(1, N_OPT)

    grid = (n_tok // TOK_BLK,)
    return pl.pallas_call(
        _fused_mlp,
        grid=grid,
        in_specs=[
            pl.BlockSpec((TOK_BLK, DIM), lambda i: (i, 0)),
            pl.BlockSpec((TOK_BLK, DIM), lambda i: (i, 0)),
            pl.BlockSpec((DIM, 2 * DIM), lambda i: (0, 0)),
            pl.BlockSpec((1, DIM), lambda i: (0, 0)),
            pl.BlockSpec((DIM, N_OPT), lambda i: (0, 0)),
            pl.BlockSpec((1, N_OPT), lambda i: (0, 0)),
        ],
        out_specs=pl.BlockSpec((TOK_BLK, N_OPT), lambda i: (i, 0)),
        out_shape=jax.ShapeDtypeStruct((n_tok, N_OPT), jnp.float32),
        scratch_shapes=[pltpu.VMEM((DIM, 2 * DIM), jnp.bfloat16)],
        compiler_params=pltpu.CompilerParams(
            dimension_semantics=("parallel",)),
    )(z, m, W1, b1r, w2t, b2r)
